# baseline (speedup 1.0000x reference)
import jax
import jax.numpy as jnp
from jax.experimental import pallas as pl
from jax.experimental.pallas import tpu as pltpu

_WINDOW = 512
_STRIDE = 256
_KSIZE = 16
_FILTERS = 32
_N = 32768
_NSLICES = 127
_OUTLEN = 497
_R = 128
_LANES = _R * _FILTERS
_K = 2 * (_R + _KSIZE)
_AROWS = _N // _R
_CHUNK = 64
_TOUT = _NSLICES * _OUTLEN


def _body(x_ref, w_ref, b_ref, o_ref, yt_ref):
    for c in range(_AROWS // _CHUNK):
        base = _CHUNK * c
        a0 = x_ref[0, pl.ds(base, _CHUNK), :]
        a1 = x_ref[0, pl.ds(base + 1, _CHUNK), 0:2 * _KSIZE]
        patch = jnp.concatenate([a0, a1], axis=1)
        y = jnp.dot(patch, w_ref[...], preferred_element_type=jnp.float32)
        y = jnp.maximum(y + b_ref[...], 0.0)
        yt = y.reshape(_CHUNK, _FILTERS, _R).transpose(1, 0, 2)
        yt_ref[:, pl.ds(_R * _CHUNK * c, _R * _CHUNK)] = yt.reshape(
            _FILTERS, _R * _CHUNK)
    for s in range(_NSLICES):
        o_ref[0, :, _OUTLEN * s:_OUTLEN * (s + 1)] = (
            yt_ref[:, _STRIDE * s:_STRIDE * s + _OUTLEN])


def kernel(x, W, b):
    B, N, C = x.shape
    xp = jnp.pad(x, ((0, 0), (0, 2 * _R), (0, 0)))
    xa = xp.reshape(B, _AROWS + 2, _R * C)
    w2 = W.reshape(_KSIZE * C, _FILTERS)
    wm = jnp.stack(
        [jnp.pad(w2, ((2 * d, _K - 2 * _KSIZE - 2 * d), (0, 0)))
         for d in range(_R)],
        axis=2).reshape(_K, _LANES)
    brep = jnp.repeat(b, _R)[None, :]

    out_t = pl.pallas_call(
        _body,
        grid=(B,),
        in_specs=[
            pl.BlockSpec((1, _AROWS + 2, 2 * _R), lambda i: (i, 0, 0)),
            pl.BlockSpec((_K, _LANES), lambda i: (0, 0)),
            pl.BlockSpec((1, _LANES), lambda i: (0, 0)),
        ],
        out_specs=pl.BlockSpec((1, _FILTERS, _TOUT), lambda i: (i, 0, 0)),
        out_shape=jax.ShapeDtypeStruct((B, _FILTERS, _TOUT), jnp.float32),
        scratch_shapes=[pltpu.VMEM((_FILTERS, _N), jnp.float32)],
        compiler_params=pltpu.CompilerParams(
            dimension_semantics=("parallel",),
            vmem_limit_bytes=56 * 1024 * 1024),
        name="netconv_fmajor",
    )(xa, wm, brep)

    return out_t.transpose(0, 2, 1)
